# baseline (speedup 1.0000x reference)
import functools

import jax
import jax.numpy as jnp
from jax.experimental import pallas as pl
from jax.experimental.pallas import tpu as pltpu

INPUT_DIM = 768
HIDDEN_DIM = 2048
TOP_K = 30
BM = 256
NB = 8192 // BM

LANES = 128
NCOLS = HIDDEN_DIM // LANES
DEPTH = 5


def _pipelined_body(x_ref, w_enc_ref, b_ref, w_dec_ref, code_ref, recon_ref,
                    h_rd_ref, h_wr_ref):
    h = h_rd_ref[...]

    cols = [h[:, i * LANES:(i + 1) * LANES] for i in range(NCOLS)]
    tmp = cols
    s_levels = []
    for d in range(DEPTH):
        m = tmp[0]
        for c in tmp[1:]:
            m = jnp.maximum(m, c)
        s_levels.append(m)
        if d < DEPTH - 1:
            tmp = [jnp.where(c >= m, -1.0, c) for c in tmp]

    def body(_, carry):
        levels = carry[:-1]
        m = jnp.max(levels[0], axis=1, keepdims=True)
        ext = levels[0] >= m
        new_levels = tuple(
            jnp.where(ext, levels[d + 1], levels[d]) for d in range(DEPTH - 1)
        ) + (jnp.where(ext, -1.0, levels[DEPTH - 1]),)
        return new_levels + (m,)

    carry = jax.lax.fori_loop(
        0, TOP_K, body,
        tuple(s_levels) + (jnp.zeros((h.shape[0], 1), jnp.float32),),
        unroll=30,
    )
    thresh = carry[-1]

    code = jnp.where(h >= thresh, h, 0.0)
    code_ref[...] = code

    recon_ref[...] = jax.lax.dot_general(
        code.astype(jnp.bfloat16), w_dec_ref[...].astype(jnp.bfloat16),
        dimension_numbers=(((1,), (1,)), ((), ())),
        preferred_element_type=jnp.float32,
    )

    hn = jax.lax.dot_general(
        x_ref[...], w_enc_ref[...],
        dimension_numbers=(((1,), (1,)), ((), ())),
        preferred_element_type=jnp.float32,
    )
    h_wr_ref[...] = jnp.maximum(hn + b_ref[...], 0.0)


def _fused_sae_kernel(x_ref, w_enc_ref, b_ref, w_dec_ref, code_ref, recon_ref,
                      h_a_ref, h_b_ref):
    i = pl.program_id(0)

    @pl.when(i % 2 == 0)
    def _():
        _pipelined_body(x_ref, w_enc_ref, b_ref, w_dec_ref, code_ref,
                        recon_ref, h_a_ref, h_b_ref)

    @pl.when(i % 2 == 1)
    def _():
        _pipelined_body(x_ref, w_enc_ref, b_ref, w_dec_ref, code_ref,
                        recon_ref, h_b_ref, h_a_ref)


@functools.partial(jax.jit, static_argnames=())
def kernel(x, W_enc, b_enc, W_dec):
    b2d = b_enc.reshape(1, HIDDEN_DIM)
    code, recon = pl.pallas_call(
        _fused_sae_kernel,
        grid=(NB + 1,),
        in_specs=[
            pl.BlockSpec((BM, INPUT_DIM), lambda i: (jnp.minimum(i, NB - 1), 0)),
            pl.BlockSpec((HIDDEN_DIM, INPUT_DIM), lambda i: (0, 0)),
            pl.BlockSpec((1, HIDDEN_DIM), lambda i: (0, 0)),
            pl.BlockSpec((INPUT_DIM, HIDDEN_DIM), lambda i: (0, 0)),
        ],
        out_specs=[
            pl.BlockSpec((BM, HIDDEN_DIM), lambda i: (jnp.maximum(i - 1, 0), 0)),
            pl.BlockSpec((BM, INPUT_DIM), lambda i: (jnp.maximum(i - 1, 0), 0)),
        ],
        out_shape=[
            jax.ShapeDtypeStruct((8192, HIDDEN_DIM), jnp.float32),
            jax.ShapeDtypeStruct((8192, INPUT_DIM), jnp.float32),
        ],
        scratch_shapes=[pltpu.VMEM((BM, HIDDEN_DIM), jnp.float32),
                        pltpu.VMEM((BM, HIDDEN_DIM), jnp.float32)],
        compiler_params=pltpu.CompilerParams(
            dimension_semantics=("arbitrary",),
        ),
    )(x, W_enc, b2d, W_dec)
    return (recon, code)


# speedup vs baseline: 1.0722x; 1.0722x over previous
import functools

import jax
import jax.numpy as jnp
from jax.experimental import pallas as pl
from jax.experimental.pallas import tpu as pltpu

INPUT_DIM = 768
HIDDEN_DIM = 2048
TOP_K = 30
BM = 256
NB = 8192 // BM

LANES = 128
NCOLS = HIDDEN_DIM // LANES
DEPTH = 5


def _pipelined_body(x_ref, w_enc_ref, b_ref, w_dec_ref, code_ref, recon_ref,
                    h_rd_ref, h_wr_ref):
    x = x_ref[...]
    w_enc = w_enc_ref[...]
    b = b_ref[...]

    def encode_chunk(j):
        hj = jax.lax.dot_general(
            x, w_enc[j * LANES:(j + 1) * LANES, :],
            dimension_numbers=(((1,), (1,)), ((), ())),
            preferred_element_type=jnp.float32,
        )
        h_wr_ref[:, j * LANES:(j + 1) * LANES] = jnp.maximum(
            hj + b[:, j * LANES:(j + 1) * LANES], 0.0)

    h = h_rd_ref[...]

    cols = [h[:, i * LANES:(i + 1) * LANES] for i in range(NCOLS)]
    tmp = cols
    s_levels = []
    for d in range(DEPTH):
        m = tmp[0]
        for c in tmp[1:]:
            m = jnp.maximum(m, c)
        s_levels.append(m)
        if d < DEPTH - 1:
            tmp = [jnp.where(c >= m, -1.0, c) for c in tmp]
        encode_chunk(d)

    levels = list(s_levels)
    thresh = None
    for k in range(TOP_K):
        m = jnp.max(levels[0], axis=1, keepdims=True)
        ext = levels[0] >= m
        levels = [
            jnp.where(ext, levels[d + 1], levels[d]) for d in range(DEPTH - 1)
        ] + [jnp.where(ext, -1.0, levels[DEPTH - 1])]
        thresh = m
        if k % 2 == 0 and DEPTH + k // 2 < NCOLS:
            encode_chunk(DEPTH + k // 2)

    code = jnp.where(h >= thresh, h, 0.0)
    code_ref[...] = code

    recon_ref[...] = jax.lax.dot_general(
        code.astype(jnp.bfloat16), w_dec_ref[...].astype(jnp.bfloat16),
        dimension_numbers=(((1,), (1,)), ((), ())),
        preferred_element_type=jnp.float32,
    )


def _fused_sae_kernel(x_ref, w_enc_ref, b_ref, w_dec_ref, code_ref, recon_ref,
                      h_a_ref, h_b_ref):
    i = pl.program_id(0)

    @pl.when(i % 2 == 0)
    def _():
        _pipelined_body(x_ref, w_enc_ref, b_ref, w_dec_ref, code_ref,
                        recon_ref, h_a_ref, h_b_ref)

    @pl.when(i % 2 == 1)
    def _():
        _pipelined_body(x_ref, w_enc_ref, b_ref, w_dec_ref, code_ref,
                        recon_ref, h_b_ref, h_a_ref)


@functools.partial(jax.jit, static_argnames=())
def kernel(x, W_enc, b_enc, W_dec):
    b2d = b_enc.reshape(1, HIDDEN_DIM)
    code, recon = pl.pallas_call(
        _fused_sae_kernel,
        grid=(NB + 1,),
        in_specs=[
            pl.BlockSpec((BM, INPUT_DIM), lambda i: (jnp.minimum(i, NB - 1), 0)),
            pl.BlockSpec((HIDDEN_DIM, INPUT_DIM), lambda i: (0, 0)),
            pl.BlockSpec((1, HIDDEN_DIM), lambda i: (0, 0)),
            pl.BlockSpec((INPUT_DIM, HIDDEN_DIM), lambda i: (0, 0)),
        ],
        out_specs=[
            pl.BlockSpec((BM, HIDDEN_DIM), lambda i: (jnp.maximum(i - 1, 0), 0)),
            pl.BlockSpec((BM, INPUT_DIM), lambda i: (jnp.maximum(i - 1, 0), 0)),
        ],
        out_shape=[
            jax.ShapeDtypeStruct((8192, HIDDEN_DIM), jnp.float32),
            jax.ShapeDtypeStruct((8192, INPUT_DIM), jnp.float32),
        ],
        scratch_shapes=[pltpu.VMEM((BM, HIDDEN_DIM), jnp.float32),
                        pltpu.VMEM((BM, HIDDEN_DIM), jnp.float32)],
        compiler_params=pltpu.CompilerParams(
            dimension_semantics=("arbitrary",),
        ),
    )(x, W_enc, b2d, W_dec)
    return (recon, code)
